# baseline (speedup 1.0000x reference)
import math

import jax
import jax.numpy as jnp
from jax import lax
from jax.experimental import pallas as pl
from jax.experimental.pallas import tpu as pltpu
from jax.experimental.pallas import tpu_sc as plsc

D = 1024
N_ROWS = 4 * 8192
NC, NS, L = 2, 16, 16
NW = NC * NS
ROWS_PER_W = N_ROWS // NW
CHUNK = 16
N_CHUNKS = ROWS_PER_W // CHUNK
NVEC = D // L
INV_SQRT_D = 1.0 / math.sqrt(D)


def _sc_body(x_hbm, lab_hbm, emb_hbm, out_hbm,
             emb_v, lab_all, xin, xout,
             in0, in1, out0, out1):
    in_sems = (in0, in1)
    out_sems = (out0, out1)
    wid = lax.axis_index("s") * NC + lax.axis_index("c")
    base = wid * ROWS_PER_W

    pltpu.sync_copy(lab_hbm.at[pl.ds(base, ROWS_PER_W)], lab_all)
    pltpu.sync_copy(emb_hbm, emb_v)
    for k in range(3):
        for c in range(NVEC):
            sl = pl.ds(c * L, L)
            emb_v[k, sl] = emb_v[k, sl] * INV_SQRT_D

    def start_in(b, i):
        pltpu.async_copy(x_hbm.at[pl.ds(base + i * CHUNK, CHUNK)],
                         xin.at[b], in_sems[b])

    def wait_in(b):
        pltpu.make_async_copy(x_hbm.at[pl.ds(0, CHUNK)],
                              xin.at[b], in_sems[b]).wait()

    def start_out(b, i):
        pltpu.async_copy(xout.at[b],
                         out_hbm.at[pl.ds(base + i * CHUNK, CHUNK)],
                         out_sems[b])

    def wait_out(b):
        pltpu.make_async_copy(xout.at[b],
                              out_hbm.at[pl.ds(0, CHUNK)],
                              out_sems[b]).wait()

    def compute(b, i):
        labv = lab_all[pl.ds(i * L, L)]
        for j in range(L):
            l = labv[j]

            def col_step(m, _):
                for t in range(8):
                    sl = pl.ds(m * 128 + t * L, L)
                    xout[b, j, sl] = xin[b, j, sl] + emb_v[l, sl]
                return 0

            lax.fori_loop(0, NVEC // 8, col_step, 0)

    start_in(0, 0)
    start_in(1, 1)

    def group_step(g, _):
        for k in range(2):
            i = g * 2 + k
            wait_in(k)

            @pl.when(i >= 2)
            def _drain():
                wait_out(k)

            compute(k, i)
            start_out(k, i)

            @pl.when(i + 2 < N_CHUNKS)
            def _prestart():
                start_in(k, i + 2)
        return 0

    lax.fori_loop(0, N_CHUNKS // 2, group_step, 0)
    wait_out(0)
    wait_out(1)


def kernel(x, role_labels, emb):
    b, s, d = x.shape
    x2 = x.reshape(b * s, d)
    lab = role_labels.astype(jnp.int32).reshape(b * s)

    mesh = plsc.VectorSubcoreMesh(core_axis_name="c", subcore_axis_name="s")
    sc_call = pl.kernel(
        _sc_body, mesh=mesh,
        out_type=jax.ShapeDtypeStruct((b * s, d), jnp.float32),
        scratch_types=[
            pltpu.VMEM((3, D), jnp.float32),
            pltpu.VMEM((ROWS_PER_W,), jnp.int32),
            pltpu.VMEM((2, CHUNK, D), jnp.float32),
            pltpu.VMEM((2, CHUNK, D), jnp.float32),
        ] + [pltpu.SemaphoreType.DMA] * 4,
    )
    out = sc_call(x2, lab, emb)
    return out.reshape(b, s, d)


# speedup vs baseline: 3.3121x; 3.3121x over previous
import math

import jax
import jax.numpy as jnp
from jax import lax
from jax.experimental import pallas as pl
from jax.experimental.pallas import tpu as pltpu
from jax.experimental.pallas import tpu_sc as plsc

D = 1024
N_ROWS = 4 * 8192
INV_SQRT_D = 1.0 / math.sqrt(D)

NC, NS, L = 2, 16, 16
NW = NC * NS
S_SC = 4096
SC_ROWS_PER_W = S_SC // NW
CHUNK = 16
NBUF = 4
SC_N_CHUNKS = SC_ROWS_PER_W // CHUNK
SC_N_GROUPS = SC_N_CHUNKS // NBUF
NVEC = D // L

ROWS_PER_BLOCK = 2048
TC_BLOCK0 = S_SC // ROWS_PER_BLOCK
TC_N_BLOCKS = (N_ROWS - S_SC) // ROWS_PER_BLOCK


def _sc_body(x_hbm, lab_hbm, emb_hbm, out_hbm,
             emb_v, lab_all, xbuf,
             in0, in1, in2, in3, out0, out1, out2, out3):
    in_sems = (in0, in1, in2, in3)
    out_sems = (out0, out1, out2, out3)
    wid = lax.axis_index("s") * NC + lax.axis_index("c")
    base = wid * SC_ROWS_PER_W

    pltpu.sync_copy(lab_hbm.at[pl.ds(base, SC_ROWS_PER_W)], lab_all)
    pltpu.sync_copy(emb_hbm, emb_v)
    for k in range(3):
        for c in range(NVEC):
            sl = pl.ds(c * L, L)
            emb_v[k, sl] = emb_v[k, sl] * INV_SQRT_D

    def start_in(b, i):
        pltpu.async_copy(x_hbm.at[pl.ds(base + i * CHUNK, CHUNK)],
                         xbuf.at[b], in_sems[b])

    def wait_in(b):
        pltpu.make_async_copy(x_hbm.at[pl.ds(0, CHUNK)],
                              xbuf.at[b], in_sems[b]).wait()

    def start_out(b, i):
        pltpu.async_copy(xbuf.at[b],
                         out_hbm.at[pl.ds(base + i * CHUNK, CHUNK)],
                         out_sems[b])

    def wait_out(b):
        pltpu.make_async_copy(xbuf.at[b],
                              out_hbm.at[pl.ds(0, CHUNK)],
                              out_sems[b]).wait()

    def compute(b, i):
        labv = lab_all[pl.ds(i * L, L)]
        for j in range(L):
            l = labv[j]

            def col_step(m, _):
                for t in range(8):
                    sl = pl.ds(m * 128 + t * L, L)
                    plsc.addupdate(xbuf.at[b, j, sl], emb_v[l, sl])
                return 0

            lax.fori_loop(0, NVEC // 8, col_step, 0)

    start_in(0, 0)
    start_in(1, 1)

    def group_step(g, _):
        for k in range(NBUF):
            i = g * NBUF + k
            b2 = (k + 2) % NBUF
            wait_in(k)
            compute(k, i)
            start_out(k, i)

            @pl.when(i + 2 < SC_N_CHUNKS)
            def _prestart():
                @pl.when(i >= 2)
                def _drain():
                    wait_out(b2)
                start_in(b2, i + 2)
        return 0

    lax.fori_loop(0, SC_N_GROUPS, group_step, 0)
    for k in range(NBUF):
        wait_out(k)


def _tc_body(lab_ref, x_ref, emb_ref, o_ref):
    lab = lab_ref[0]
    r = lab.shape[-1]
    ohT = (jax.lax.broadcasted_iota(jnp.int32, (3, r), 0) == lab).astype(jnp.float32)
    rows = jax.lax.dot_general(
        ohT, emb_ref[...],
        dimension_numbers=(((0,), (0,)), ((), ())),
        preferred_element_type=jnp.float32,
    )
    o_ref[...] = x_ref[...] + rows * INV_SQRT_D


def kernel(x, role_labels, emb):
    b, s, d = x.shape
    n_rows = b * s
    x2 = x.reshape(n_rows, d)
    lab = role_labels.astype(jnp.int32).reshape(n_rows)

    mesh = plsc.VectorSubcoreMesh(core_axis_name="c", subcore_axis_name="s")
    sc_call = pl.kernel(
        _sc_body, mesh=mesh,
        out_type=jax.ShapeDtypeStruct((S_SC, d), jnp.float32),
        scratch_types=[
            pltpu.VMEM((3, D), jnp.float32),
            pltpu.VMEM((SC_ROWS_PER_W,), jnp.int32),
            pltpu.VMEM((NBUF, CHUNK, D), jnp.float32),
        ] + [pltpu.SemaphoreType.DMA] * 8,
    )
    sc_out = sc_call(x2, lab, emb)

    g = n_rows // ROWS_PER_BLOCK
    lab3 = lab.reshape(g, 1, ROWS_PER_BLOCK)
    tc_out = pl.pallas_call(
        _tc_body,
        grid=(TC_N_BLOCKS,),
        in_specs=[
            pl.BlockSpec((1, 1, ROWS_PER_BLOCK), lambda i: (i + TC_BLOCK0, 0, 0)),
            pl.BlockSpec((ROWS_PER_BLOCK, d), lambda i: (i + TC_BLOCK0, 0)),
            pl.BlockSpec((3, d), lambda i: (0, 0)),
        ],
        out_specs=pl.BlockSpec((ROWS_PER_BLOCK, d), lambda i: (i + TC_BLOCK0, 0)),
        out_shape=jax.ShapeDtypeStruct((n_rows, d), jnp.float32),
    )(lab3, x2, emb)

    out = lax.dynamic_update_slice(tc_out, sc_out, (0, 0))
    return out.reshape(b, s, d)


# speedup vs baseline: 3.4421x; 1.0392x over previous
import math

import jax
import jax.numpy as jnp
from jax import lax
from jax.experimental import pallas as pl
from jax.experimental.pallas import tpu as pltpu
from jax.experimental.pallas import tpu_sc as plsc

D = 1024
N_ROWS = 4 * 8192
INV_SQRT_D = 1.0 / math.sqrt(D)

NC, NS, L = 2, 16, 16
NW = NC * NS
S_SC = 2048
SC_ROWS_PER_W = S_SC // NW
CHUNK = 16
NBUF = 4
SC_N_CHUNKS = SC_ROWS_PER_W // CHUNK
SC_N_GROUPS = SC_N_CHUNKS // NBUF
NVEC = D // L

ROWS_PER_BLOCK = 2048
TC_BLOCK0 = S_SC // ROWS_PER_BLOCK
TC_N_BLOCKS = (N_ROWS - S_SC) // ROWS_PER_BLOCK


def _sc_body(x_hbm, lab_hbm, emb_hbm, out_hbm,
             emb_v, lab_all, xbuf,
             in0, in1, in2, in3, out0, out1, out2, out3):
    in_sems = (in0, in1, in2, in3)
    out_sems = (out0, out1, out2, out3)
    wid = lax.axis_index("s") * NC + lax.axis_index("c")
    base = wid * SC_ROWS_PER_W

    pltpu.sync_copy(lab_hbm.at[pl.ds(base, SC_ROWS_PER_W)], lab_all)
    pltpu.sync_copy(emb_hbm, emb_v)
    for k in range(3):
        for c in range(NVEC):
            sl = pl.ds(c * L, L)
            emb_v[k, sl] = emb_v[k, sl] * INV_SQRT_D

    def start_in(b, i):
        pltpu.async_copy(x_hbm.at[pl.ds(base + i * CHUNK, CHUNK)],
                         xbuf.at[b], in_sems[b])

    def wait_in(b):
        pltpu.make_async_copy(x_hbm.at[pl.ds(0, CHUNK)],
                              xbuf.at[b], in_sems[b]).wait()

    def start_out(b, i):
        pltpu.async_copy(xbuf.at[b],
                         out_hbm.at[pl.ds(base + i * CHUNK, CHUNK)],
                         out_sems[b])

    def wait_out(b):
        pltpu.make_async_copy(xbuf.at[b],
                              out_hbm.at[pl.ds(0, CHUNK)],
                              out_sems[b]).wait()

    def compute(b, i):
        labv = lab_all[pl.ds(i * L, L)]
        for j in range(L):
            l = labv[j]

            def col_step(m, _):
                for t in range(8):
                    sl = pl.ds(m * 128 + t * L, L)
                    plsc.addupdate(xbuf.at[b, j, sl], emb_v[l, sl])
                return 0

            lax.fori_loop(0, NVEC // 8, col_step, 0)

    start_in(0, 0)
    start_in(1, 1)

    def group_step(g, _):
        for k in range(NBUF):
            i = g * NBUF + k
            b2 = (k + 2) % NBUF
            wait_in(k)
            compute(k, i)
            start_out(k, i)

            @pl.when(i + 2 < SC_N_CHUNKS)
            def _prestart():
                @pl.when(i >= 2)
                def _drain():
                    wait_out(b2)
                start_in(b2, i + 2)
        return 0

    lax.fori_loop(0, SC_N_GROUPS, group_step, 0)
    for k in range(NBUF):
        wait_out(k)


def _tc_body(lab_ref, x_ref, emb_ref, o_ref):
    lab = lab_ref[0]
    r = lab.shape[-1]
    ohT = (jax.lax.broadcasted_iota(jnp.int32, (3, r), 0) == lab).astype(jnp.float32)
    rows = jax.lax.dot_general(
        ohT, emb_ref[...],
        dimension_numbers=(((0,), (0,)), ((), ())),
        preferred_element_type=jnp.float32,
    )
    o_ref[...] = x_ref[...] + rows * INV_SQRT_D


def kernel(x, role_labels, emb):
    b, s, d = x.shape
    n_rows = b * s
    x2 = x.reshape(n_rows, d)
    lab = role_labels.astype(jnp.int32).reshape(n_rows)

    mesh = plsc.VectorSubcoreMesh(core_axis_name="c", subcore_axis_name="s")
    sc_call = pl.kernel(
        _sc_body, mesh=mesh,
        out_type=jax.ShapeDtypeStruct((S_SC, d), jnp.float32),
        scratch_types=[
            pltpu.VMEM((3, D), jnp.float32),
            pltpu.VMEM((SC_ROWS_PER_W,), jnp.int32),
            pltpu.VMEM((NBUF, CHUNK, D), jnp.float32),
        ] + [pltpu.SemaphoreType.DMA] * 8,
    )
    sc_out = sc_call(x2, lab, emb)

    g = n_rows // ROWS_PER_BLOCK
    lab3 = lab.reshape(g, 1, ROWS_PER_BLOCK)
    tc_out = pl.pallas_call(
        _tc_body,
        grid=(TC_N_BLOCKS,),
        in_specs=[
            pl.BlockSpec((1, 1, ROWS_PER_BLOCK), lambda i: (i + TC_BLOCK0, 0, 0)),
            pl.BlockSpec((ROWS_PER_BLOCK, d), lambda i: (i + TC_BLOCK0, 0)),
            pl.BlockSpec((3, d), lambda i: (0, 0)),
        ],
        out_specs=pl.BlockSpec((ROWS_PER_BLOCK, d), lambda i: (i + TC_BLOCK0, 0)),
        out_shape=jax.ShapeDtypeStruct((n_rows, d), jnp.float32),
    )(lab3, x2, emb)

    out = lax.dynamic_update_slice(tc_out, sc_out, (0, 0))
    return out.reshape(b, s, d)
